# baseline (speedup 1.0000x reference)
import functools

import jax
import jax.numpy as jnp
from jax import lax
from jax.experimental import pallas as pl
from jax.experimental.pallas import tpu as pltpu
from jax.experimental.pallas import tpu_sc as plsc

_C = 72
_NS = 16


def _make_spmm_pair(nacc, chunks):
    zrows = nacc // _NS
    mesh = plsc.VectorSubcoreMesh(core_axis_name="c", subcore_axis_name="s")

    @functools.partial(
        pl.kernel,
        mesh=mesh,
        out_type=jax.ShapeDtypeStruct((2, nacc, 128), jnp.float32),
        scratch_types=[
            pltpu.VMEM((5, _C), jnp.int32),
            pltpu.VMEM((5, _C), jnp.int32),
            pltpu.VMEM((5, _C, 128), jnp.float32),
            pltpu.VMEM_SHARED((nacc, 128), jnp.float32),
        ] + [pltpu.SemaphoreType.DMA] * 15,
    )
    def spmm_pair(h_hbm, src_hbm, dst_hbm, zeros_hbm, out_hbm,
                  srcv, dstv, rows, accum, *sems):
        cid = lax.axis_index("c")
        sid = lax.axis_index("s")
        semis = sems[0:5]
        semid = sems[5:10]
        semg = sems[10:15]
        pltpu.sync_copy(zeros_hbm, accum.at[pl.ds(sid * zrows, zrows)])
        plsc.subcore_barrier()

        def idx_start(i, r):
            pltpu.async_copy(src_hbm.at[cid, sid, i], srcv.at[r], semis[r])
            pltpu.async_copy(dst_hbm.at[cid, sid, i], dstv.at[r], semid[r])

        def idx_wait(r):
            pltpu.make_async_copy(
                src_hbm.at[cid, sid, 0], srcv.at[r], semis[r]).wait()

        def gather_start(r):
            pltpu.async_copy(h_hbm.at[srcv.at[r]], rows.at[r], semg[r])

        def gather_wait(r):
            pltpu.make_async_copy(
                h_hbm.at[srcv.at[r]], rows.at[r], semg[r]).wait()

        def scatter(r):
            pltpu.make_async_copy(
                src_hbm.at[cid, sid, 0], dstv.at[r], semid[r]).wait()
            pltpu.sync_copy(rows.at[r], accum.at[dstv.at[r]], add=True)

        for r0 in range(5):
            idx_start(r0, r0)
        for r0 in range(4):
            idx_wait(r0)
            gather_start(r0)

        def step(g, carry):
            i0 = 5 * g
            for r in range(5):
                i = i0 + r
                n4 = (r + 4) % 5
                gather_wait(r)

                @pl.when(i + 4 < chunks)
                def _(i=i, n4=n4):
                    idx_wait(n4)
                    gather_start(n4)

                scatter(r)

                @pl.when(i + 5 < chunks)
                def _(i=i, r=r):
                    idx_start(i + 5, r)

            return carry

        lax.fori_loop(0, chunks // 5, step, 0)
        plsc.subcore_barrier()
        pltpu.sync_copy(accum.at[pl.ds(sid * zrows, zrows)],
                        out_hbm.at[cid, pl.ds(sid * zrows, zrows)])

    return spmm_pair


def _dot(a, b):
    return jnp.dot(a, b, preferred_element_type=jnp.float32)


def _stats_body(n, a_ref, cs_ref, csq_ref):
    for half in (0, 1):
        a = a_ref[half, :, :]
        bad = a[n:n + 1, :]
        cs = jnp.sum(a, axis=0, keepdims=True) - bad
        csq = jnp.sum(a * a, axis=0, keepdims=True) - bad * bad
        cs_ref[half, :, :] = jnp.broadcast_to(cs, (8, 128))
        csq_ref[half, :, :] = jnp.broadcast_to(csq, (8, 128))


def _pairnorm_blocks(n, a, cs, csq):
    mu = cs * (1.0 / n)
    t = csq * (1.0 / n) - mu * mu
    rn_a = jnp.sqrt(1e-6 + jnp.sum(t[:, :64]))
    rn_b = jnp.sqrt(1e-6 + jnp.sum(t[:, 64:]))
    c = a - mu
    return jnp.tanh(c[:, :64] / rn_a), jnp.tanh(c[:, 64:] / rn_b)


def _tc0_body(x_ref, ws1_ref, wc_ref, ws2_ref, out_ref):
    x = x_ref[...]
    hc = _dot(x, wc_ref[...])
    out_ref[0, :, :] = jnp.concatenate([_dot(x, ws1_ref[...]), hc], axis=1)
    out_ref[1, :, :] = jnp.concatenate([_dot(x, ws2_ref[...]), hc], axis=1)


def _tc1_body(n, a_ref, cs_ref, csq_ref, x_ref,
              ws1x_ref, ws1b_ref, wcx_ref, wcb_ref, ws2x_ref, ws2b_ref,
              h_ref, b0_ref):
    x = x_ref[...]
    side = ((ws1x_ref, ws1b_ref), (ws2x_ref, ws2b_ref))
    for half in (0, 1):
        wx, wb = side[half]
        blk_a, blk_b = _pairnorm_blocks(
            n, a_ref[half, :, :], cs_ref[half, 0:1, :], csq_ref[half, 0:1, :])
        h_a = _dot(x, wx[...]) + _dot(blk_a, wb[...])
        h_b = _dot(x, wcx_ref[...]) + _dot(blk_b, wcb_ref[...])
        h_ref[half, :, :] = jnp.concatenate([h_a, h_b], axis=1)
        b0_ref[half, :, :] = jnp.concatenate([blk_a, blk_b], axis=1)


def _tc2_body(n, a_ref, cs_ref, csq_ref, x_ref, b0_ref,
              ws1x_ref, ws1a_ref, ws1b_ref, wcx_ref, wca_ref, wcb_ref,
              ws2x_ref, ws2a_ref, ws2b_ref, h_ref):
    x = x_ref[...]
    side = ((ws1x_ref, ws1a_ref, ws1b_ref), (ws2x_ref, ws2a_ref, ws2b_ref))
    for half in (0, 1):
        wx, wa, wb = side[half]
        blk_a, blk_b = _pairnorm_blocks(
            n, a_ref[half, :, :], cs_ref[half, 0:1, :], csq_ref[half, 0:1, :])
        b0_a = b0_ref[half, :, :64]
        b0_b = b0_ref[half, :, 64:]
        h_a = _dot(x, wx[...]) + _dot(b0_a, wa[...]) + _dot(blk_a, wb[...])
        h_b = (_dot(x, wcx_ref[...]) + _dot(b0_b, wca_ref[...])
               + _dot(blk_b, wcb_ref[...]))
        h_ref[half, :, :] = jnp.concatenate([h_a, h_b], axis=1)


def _tc3_body(a_ref, bo1_ref, boc_ref, bo2_ref,
              aw1_ref, ab1_ref, aw2_ref, mw_ref, mb_ref,
              out_ref, beta_ref, emb1_ref, com1_ref, com2_ref, emb2_ref):
    def norm_rows(o):
        nrm = jnp.sqrt(jnp.sum(o * o, axis=1, keepdims=True))
        return o / jnp.maximum(nrm, 1e-12)

    emb1 = norm_rows(a_ref[0, :, :64] + bo1_ref[...])
    com1 = norm_rows(a_ref[0, :, 64:] + boc_ref[...])
    emb2 = norm_rows(a_ref[1, :, :64] + bo2_ref[...])
    com2 = norm_rows(a_ref[1, :, 64:] + boc_ref[...])
    xcom = (com1 + com2) * 0.5

    aw1 = aw1_ref[...]
    ab1 = ab1_ref[...]
    aw2 = aw2_ref[...]
    scores = jnp.concatenate(
        [_dot(jnp.tanh(_dot(v, aw1) + ab1), aw2) for v in (emb1, emb2, xcom)],
        axis=1)
    m = jnp.max(scores, axis=1, keepdims=True)
    ex = jnp.exp(scores - m)
    beta = ex / jnp.sum(ex, axis=1, keepdims=True)

    emb = beta[:, 0:1] * emb1 + beta[:, 1:2] * emb2 + beta[:, 2:3] * xcom
    logits = _dot(emb, mw_ref[...]) + mb_ref[...]
    lm = jnp.max(logits, axis=1, keepdims=True)
    le = jnp.exp(logits - lm)
    out_ref[...] = le / jnp.sum(le, axis=1, keepdims=True)
    beta_ref[...] = beta
    emb1_ref[...] = emb1
    com1_ref[...] = com1
    com2_ref[...] = com2
    emb2_ref[...] = emb2


def _full_spec(shape):
    nd = len(shape)
    return pl.BlockSpec(shape, lambda i, _nd=nd: (0,) * _nd)


def _rows_spec(bs, width):
    return pl.BlockSpec((bs, width), lambda i: (i, 0))


def _half_rows_spec(bs, width):
    return pl.BlockSpec((2, bs, width), lambda i: (0, i, 0))


def kernel(x, params, padj, fadj):
    n, nfeat = x.shape
    e = padj.shape[1]
    f32 = jnp.float32

    chunks = -(-e // (_NS * _C))
    chunks = 5 * (-(-chunks // 5))
    t = chunks * _C
    tot = _NS * t
    nacc = _NS * 8 * (-(-(n + 1) // (_NS * 8)))
    bs = nacc // 8
    grid = (8,)

    def prep(src, dst, off):
        s = jnp.pad(src + off, (0, tot - e)).reshape(_NS, chunks, _C)
        d = jnp.pad(dst, (0, tot - e), constant_values=n).reshape(
            _NS, chunks, _C)
        return s, d

    sp, dp = prep(padj[0], padj[1], 0)
    sf, df = prep(fadj[0], fadj[1], n)
    src_all = jnp.stack([sp, sf])
    dst_all = jnp.stack([dp, df])
    zeros = jnp.zeros((nacc // _NS, 128), f32)

    spmm_pair = _make_spmm_pair(nacc, chunks)

    p1, p2, pc = params["sgcn1"], params["sgcn2"], params["cgcn"]
    nh = p1["ws"][1].shape[0] - nfeat
    w64 = _full_spec((nfeat, 64))
    h64 = _full_spec((nh, 64))
    stat_spec = _full_spec((2, 8, 128))
    stat_shape = jax.ShapeDtypeStruct((2, 8, 128), f32)

    def stats(a):
        return pl.pallas_call(
            functools.partial(_stats_body, n),
            out_shape=[stat_shape, stat_shape],
        )(a)

    h0 = pl.pallas_call(
        _tc0_body,
        grid=grid,
        in_specs=[_rows_spec(bs, nfeat), w64, w64, w64],
        out_specs=_half_rows_spec(bs, 128),
        out_shape=jax.ShapeDtypeStruct((2, n, 128), f32),
    )(x, p1["ws"][0], pc["ws"][0], p2["ws"][0])

    a0 = spmm_pair(h0.reshape(2 * n, 128), src_all, dst_all, zeros)

    cs0, csq0 = stats(a0)
    h1, b0 = pl.pallas_call(
        functools.partial(_tc1_body, n),
        grid=grid,
        in_specs=[_half_rows_spec(bs, 128), stat_spec, stat_spec,
                  _rows_spec(bs, nfeat), w64, h64, w64, h64, w64, h64],
        out_specs=[_half_rows_spec(bs, 128), _half_rows_spec(bs, 128)],
        out_shape=[jax.ShapeDtypeStruct((2, n, 128), f32),
                   jax.ShapeDtypeStruct((2, n, 128), f32)],
    )(a0, cs0, csq0, x,
      p1["ws"][1][:nfeat], p1["ws"][1][nfeat:],
      pc["ws"][1][:nfeat], pc["ws"][1][nfeat:],
      p2["ws"][1][:nfeat], p2["ws"][1][nfeat:])

    a1 = spmm_pair(h1.reshape(2 * n, 128), src_all, dst_all, zeros)

    cs1, csq1 = stats(a1)
    h2 = pl.pallas_call(
        functools.partial(_tc2_body, n),
        grid=grid,
        in_specs=[_half_rows_spec(bs, 128), stat_spec, stat_spec,
                  _rows_spec(bs, nfeat), _half_rows_spec(bs, 128),
                  w64, h64, h64, w64, h64, h64, w64, h64, h64],
        out_specs=_half_rows_spec(bs, 128),
        out_shape=jax.ShapeDtypeStruct((2, n, 128), f32),
    )(a1, cs1, csq1, x, b0,
      p1["w_out"][:nfeat], p1["w_out"][nfeat:nfeat + nh],
      p1["w_out"][nfeat + nh:],
      pc["w_out"][:nfeat], pc["w_out"][nfeat:nfeat + nh],
      pc["w_out"][nfeat + nh:],
      p2["w_out"][:nfeat], p2["w_out"][nfeat:nfeat + nh],
      p2["w_out"][nfeat + nh:])

    a2 = spmm_pair(h2.reshape(2 * n, 128), src_all, dst_all, zeros)

    nclass = params["mlp_w"].shape[1]
    out, beta, emb1, com1, com2, emb2 = pl.pallas_call(
        _tc3_body,
        grid=grid,
        in_specs=[_half_rows_spec(bs, 128),
                  _full_spec((64,)), _full_spec((64,)), _full_spec((64,)),
                  _full_spec((64, 2)), _full_spec((2,)), _full_spec((2, 1)),
                  _full_spec((64, nclass)), _full_spec((nclass,))],
        out_specs=[_rows_spec(bs, nclass), _rows_spec(bs, 3),
                   _rows_spec(bs, 64), _rows_spec(bs, 64),
                   _rows_spec(bs, 64), _rows_spec(bs, 64)],
        out_shape=[jax.ShapeDtypeStruct((n, nclass), f32),
                   jax.ShapeDtypeStruct((n, 3), f32),
                   jax.ShapeDtypeStruct((n, 64), f32),
                   jax.ShapeDtypeStruct((n, 64), f32),
                   jax.ShapeDtypeStruct((n, 64), f32),
                   jax.ShapeDtypeStruct((n, 64), f32)],
    )(a2, p1["b_out"], pc["b_out"], p2["b_out"],
      params["att_w1"], params["att_b1"], params["att_w2"],
      params["mlp_w"], params["mlp_b"])

    shift_loss = jnp.zeros((1,), f32)
    return (out, shift_loss, beta.reshape(n, 3, 1), emb1, com1, com2, emb2)


# speedup vs baseline: 1.3063x; 1.3063x over previous
import functools

import jax
import jax.numpy as jnp
import numpy as np
from jax import lax
from jax.experimental import pallas as pl
from jax.experimental.pallas import tpu as pltpu
from jax.experimental.pallas import tpu_sc as plsc

_C = 88
_R = 4
_NS = 16

def _make_spmm_pair(nacc, chunks):
    zrows = nacc // _NS
    mesh = plsc.VectorSubcoreMesh(core_axis_name="c", subcore_axis_name="s")

    @functools.partial(
        pl.kernel,
        mesh=mesh,
        out_type=jax.ShapeDtypeStruct((2, nacc, 128), jnp.float32),
        scratch_types=[
            pltpu.VMEM((2, _C), jnp.int32),
            pltpu.VMEM((2, _C), jnp.int32),
            pltpu.VMEM((2, _C), jnp.int32),
            pltpu.VMEM((2, _C), jnp.int32),
            pltpu.VMEM((_C, 128), jnp.float32),
            pltpu.VMEM((_C, 128), jnp.float32),
            pltpu.VMEM((_C, 128), jnp.float32),
            pltpu.VMEM((_C, 128), jnp.float32),
            pltpu.VMEM_SHARED((nacc, 128), jnp.float32),
        ] + [pltpu.SemaphoreType.DMA] * (2 * _R),
    )
    def spmm_pair(h_hbm, sd_hbm, zeros_hbm, out_hbm,
                  sd0, sd1, sd2, sd3, rw0, rw1, rw2, rw3, accum, *sems):
        cid = lax.axis_index("c")
        sid = lax.axis_index("s")
        sdv = (sd0, sd1, sd2, sd3)
        rows = (rw0, rw1, rw2, rw3)
        semi = sems[0:_R]
        semg = sems[_R:2 * _R]
        pltpu.sync_copy(zeros_hbm, accum.at[pl.ds(sid * zrows, zrows)])
        plsc.subcore_barrier()

        def idx_start(i, r):
            pltpu.async_copy(sd_hbm.at[cid, sid, i], sdv[r], semi[r])

        def idx_wait(r):
            pltpu.make_async_copy(
                sd_hbm.at[cid, sid, 0], sdv[r], semi[r]).wait()

        def gather_start(r):
            pltpu.async_copy(h_hbm.at[sdv[r].at[0]], rows[r], semg[r])

        def gather_wait(r):
            pltpu.make_async_copy(
                h_hbm.at[sdv[r].at[0]], rows[r], semg[r]).wait()

        def scatter(r):
            pltpu.sync_copy(rows[r], accum.at[sdv[r].at[1]], add=True)

        for r0 in range(_R):
            idx_start(r0, r0)
        for r0 in range(_R - 1):
            idx_wait(r0)
            gather_start(r0)

        def step(g, carry):
            i0 = _R * g
            for r in range(_R):
                i = i0 + r
                nx = (r + _R - 1) % _R
                gather_wait(r)

                @pl.when(i + _R - 1 < chunks)
                def _(i=i, nx=nx):
                    idx_wait(nx)
                    gather_start(nx)

                scatter(r)

                @pl.when(i + _R < chunks)
                def _(i=i, r=r):
                    idx_start(i + _R, r)

            return carry

        lax.fori_loop(0, chunks // _R, step, 0)
        plsc.subcore_barrier()
        pltpu.sync_copy(accum.at[pl.ds(sid * zrows, zrows)],
                        out_hbm.at[cid, pl.ds(sid * zrows, zrows)])

    return spmm_pair


def _dot(a, b):
    return jnp.dot(a, b, preferred_element_type=jnp.float32)


def _stats_body(n, a_ref, cs_ref, csq_ref):
    for half in (0, 1):
        a = a_ref[half, :, :]
        bad = a[n:n + 1, :]
        cs = jnp.sum(a, axis=0, keepdims=True) - bad
        csq = jnp.sum(a * a, axis=0, keepdims=True) - bad * bad
        cs_ref[half, :, :] = jnp.broadcast_to(cs, (8, 128))
        csq_ref[half, :, :] = jnp.broadcast_to(csq, (8, 128))


def _pairnorm_blocks(n, a, cs, csq):
    mu = cs * (1.0 / n)
    t = csq * (1.0 / n) - mu * mu
    rn_a = jnp.sqrt(1e-6 + jnp.sum(t[:, :64]))
    rn_b = jnp.sqrt(1e-6 + jnp.sum(t[:, 64:]))
    c = a - mu
    return jnp.tanh(c[:, :64] / rn_a), jnp.tanh(c[:, 64:] / rn_b)


def _tc0_body(x_ref, w0_ref, w1_ref, out_ref):
    x = x_ref[...]
    out_ref[0, :, :] = _dot(x, w0_ref[...])
    out_ref[1, :, :] = _dot(x, w1_ref[...])


def _tc1_body(n, a_ref, cs_ref, csq_ref, x_ref,
              wx0_ref, wa0_ref, wb0_ref, wx1_ref, wa1_ref, wb1_ref,
              h_ref, b0_ref):
    x = x_ref[...]
    side = ((wx0_ref, wa0_ref, wb0_ref), (wx1_ref, wa1_ref, wb1_ref))
    for half in (0, 1):
        wx, wa, wb = side[half]
        blk_a, blk_b = _pairnorm_blocks(
            n, a_ref[half, :, :], cs_ref[half, 0:1, :], csq_ref[half, 0:1, :])
        h = _dot(x, wx[...]) + _dot(blk_a, wa[...]) + _dot(blk_b, wb[...])
        h_ref[half, :, :] = h
        b0_ref[half, :, :] = jnp.concatenate([blk_a, blk_b], axis=1)


def _tc2_body(n, a_ref, cs_ref, csq_ref, x_ref, b0_ref,
              wx0_ref, wa00_ref, wb00_ref, wa10_ref, wb10_ref,
              wx1_ref, wa01_ref, wb01_ref, wa11_ref, wb11_ref, h_ref):
    x = x_ref[...]
    side = ((wx0_ref, wa00_ref, wb00_ref, wa10_ref, wb10_ref),
            (wx1_ref, wa01_ref, wb01_ref, wa11_ref, wb11_ref))
    for half in (0, 1):
        wx, wa0, wb0, wa1, wb1 = side[half]
        blk_a, blk_b = _pairnorm_blocks(
            n, a_ref[half, :, :], cs_ref[half, 0:1, :], csq_ref[half, 0:1, :])
        b0_a = b0_ref[half, :, :64]
        b0_b = b0_ref[half, :, 64:]
        h = (_dot(x, wx[...]) + _dot(b0_a, wa0[...]) + _dot(b0_b, wb0[...])
             + _dot(blk_a, wa1[...]) + _dot(blk_b, wb1[...]))
        h_ref[half, :, :] = h


def _tc3_body(a_ref, bo1_ref, boc_ref, bo2_ref,
              aw1_ref, ab1_ref, aw2_ref, mw_ref, mb_ref,
              out_ref, beta_ref, emb1_ref, com1_ref, com2_ref, emb2_ref):
    def norm_rows(o):
        nrm = jnp.sqrt(jnp.sum(o * o, axis=1, keepdims=True))
        return o / jnp.maximum(nrm, 1e-12)

    emb1 = norm_rows(a_ref[0, :, :64] + bo1_ref[...])
    com1 = norm_rows(a_ref[0, :, 64:] + boc_ref[...])
    emb2 = norm_rows(a_ref[1, :, :64] + bo2_ref[...])
    com2 = norm_rows(a_ref[1, :, 64:] + boc_ref[...])
    xcom = (com1 + com2) * 0.5

    aw1 = aw1_ref[...]
    ab1 = ab1_ref[...]
    aw2 = aw2_ref[...]
    scores = jnp.concatenate(
        [_dot(jnp.tanh(_dot(v, aw1) + ab1), aw2) for v in (emb1, emb2, xcom)],
        axis=1)
    m = jnp.max(scores, axis=1, keepdims=True)
    ex = jnp.exp(scores - m)
    beta = ex / jnp.sum(ex, axis=1, keepdims=True)

    emb = beta[:, 0:1] * emb1 + beta[:, 1:2] * emb2 + beta[:, 2:3] * xcom
    logits = _dot(emb, mw_ref[...]) + mb_ref[...]
    lm = jnp.max(logits, axis=1, keepdims=True)
    le = jnp.exp(logits - lm)
    out_ref[...] = le / jnp.sum(le, axis=1, keepdims=True)
    beta_ref[...] = beta
    emb1_ref[...] = emb1
    com1_ref[...] = com1
    com2_ref[...] = com2
    emb2_ref[...] = emb2


def _full_spec(shape):
    nd = len(shape)
    return pl.BlockSpec(shape, lambda i, _nd=nd: (0,) * _nd)


def _rows_spec(bs, width):
    return pl.BlockSpec((bs, width), lambda i: (i, 0))


def _half_rows_spec(bs, width):
    return pl.BlockSpec((2, bs, width), lambda i: (0, i, 0))


def kernel(x, params, padj, fadj):
    n, nfeat = x.shape
    e = padj.shape[1]
    f32 = jnp.float32

    chunks = -(-e // (_NS * _C))
    chunks = _R * (-(-chunks // _R))
    t = chunks * _C
    tot = _NS * t
    nacc = _NS * 8 * (-(-(n + 1) // (_NS * 8)))
    bs = nacc // 8
    grid = (8,)

    def prep(src, dst, off):
        s = jnp.pad(src + off, (0, tot - e)).reshape(_NS, chunks, 1, _C)
        d = jnp.pad(dst, (0, tot - e), constant_values=n).reshape(
            _NS, chunks, 1, _C)
        return jnp.concatenate([s, d], axis=2)

    sd_all = jnp.stack([prep(padj[0], padj[1], 0),
                        prep(fadj[0], fadj[1], n)])
    zeros = jnp.zeros((nacc // _NS, 128), f32)

    spmm_pair = _make_spmm_pair(nacc, chunks)

    p1, p2, pc = params["sgcn1"], params["sgcn2"], params["cgcn"]
    nh = p1["ws"][1].shape[0] - nfeat
    z64 = jnp.zeros((nh, 64), f32)

    def comb(wa, wb):
        return jnp.concatenate([wa, wb], axis=1)

    w128 = _full_spec((nfeat, 128))
    w64 = _full_spec((nh, 128))
    stat_spec = _full_spec((2, 8, 128))
    stat_shape = jax.ShapeDtypeStruct((2, 8, 128), f32)

    def stats(a):
        return pl.pallas_call(
            functools.partial(_stats_body, n),
            out_shape=[stat_shape, stat_shape],
        )(a)

    h0 = pl.pallas_call(
        _tc0_body,
        grid=grid,
        in_specs=[_rows_spec(bs, nfeat), w128, w128],
        out_specs=_half_rows_spec(bs, 128),
        out_shape=jax.ShapeDtypeStruct((2, n, 128), f32),
    )(x, comb(p1["ws"][0], pc["ws"][0]), comb(p2["ws"][0], pc["ws"][0]))

    a0 = spmm_pair(h0.reshape(2 * n, 128), sd_all, zeros)

    cs0, csq0 = stats(a0)
    h1, b0 = pl.pallas_call(
        functools.partial(_tc1_body, n),
        grid=grid,
        in_specs=[_half_rows_spec(bs, 128), stat_spec, stat_spec,
                  _rows_spec(bs, nfeat), w128, w64, w64, w128, w64, w64],
        out_specs=[_half_rows_spec(bs, 128), _half_rows_spec(bs, 128)],
        out_shape=[jax.ShapeDtypeStruct((2, n, 128), f32),
                   jax.ShapeDtypeStruct((2, n, 128), f32)],
    )(a0, cs0, csq0, x,
      comb(p1["ws"][1][:nfeat], pc["ws"][1][:nfeat]),
      comb(p1["ws"][1][nfeat:], z64), comb(z64, pc["ws"][1][nfeat:]),
      comb(p2["ws"][1][:nfeat], pc["ws"][1][:nfeat]),
      comb(p2["ws"][1][nfeat:], z64), comb(z64, pc["ws"][1][nfeat:]))

    a1 = spmm_pair(h1.reshape(2 * n, 128), sd_all, zeros)

    cs1, csq1 = stats(a1)
    h2 = pl.pallas_call(
        functools.partial(_tc2_body, n),
        grid=grid,
        in_specs=[_half_rows_spec(bs, 128), stat_spec, stat_spec,
                  _rows_spec(bs, nfeat), _half_rows_spec(bs, 128),
                  w128, w64, w64, w64, w64, w128, w64, w64, w64, w64],
        out_specs=_half_rows_spec(bs, 128),
        out_shape=jax.ShapeDtypeStruct((2, n, 128), f32),
    )(a1, cs1, csq1, x, b0,
      comb(p1["w_out"][:nfeat], pc["w_out"][:nfeat]),
      comb(p1["w_out"][nfeat:nfeat + nh], z64),
      comb(z64, pc["w_out"][nfeat:nfeat + nh]),
      comb(p1["w_out"][nfeat + nh:], z64),
      comb(z64, pc["w_out"][nfeat + nh:]),
      comb(p2["w_out"][:nfeat], pc["w_out"][:nfeat]),
      comb(p2["w_out"][nfeat:nfeat + nh], z64),
      comb(z64, pc["w_out"][nfeat:nfeat + nh]),
      comb(p2["w_out"][nfeat + nh:], z64),
      comb(z64, pc["w_out"][nfeat + nh:]))

    a2 = spmm_pair(h2.reshape(2 * n, 128), sd_all, zeros)

    nclass = params["mlp_w"].shape[1]
    out, beta, emb1, com1, com2, emb2 = pl.pallas_call(
        _tc3_body,
        grid=grid,
        in_specs=[_half_rows_spec(bs, 128),
                  _full_spec((64,)), _full_spec((64,)), _full_spec((64,)),
                  _full_spec((64, 2)), _full_spec((2,)), _full_spec((2, 1)),
                  _full_spec((64, nclass)), _full_spec((nclass,))],
        out_specs=[_rows_spec(bs, nclass), _rows_spec(bs, 3),
                   _rows_spec(bs, 64), _rows_spec(bs, 64),
                   _rows_spec(bs, 64), _rows_spec(bs, 64)],
        out_shape=[jax.ShapeDtypeStruct((n, nclass), f32),
                   jax.ShapeDtypeStruct((n, 3), f32),
                   jax.ShapeDtypeStruct((n, 64), f32),
                   jax.ShapeDtypeStruct((n, 64), f32),
                   jax.ShapeDtypeStruct((n, 64), f32),
                   jax.ShapeDtypeStruct((n, 64), f32)],
    )(a2, p1["b_out"], pc["b_out"], p2["b_out"],
      params["att_w1"], params["att_b1"], params["att_w2"],
      params["mlp_w"], params["mlp_b"])

    shift_loss = jnp.zeros((1,), f32)
    return (out, shift_loss, beta.reshape(n, 3, 1), emb1, com1, com2, emb2)


# speedup vs baseline: 1.3551x; 1.0374x over previous
import functools

import jax
import jax.numpy as jnp
import numpy as np
from jax import lax
from jax.experimental import pallas as pl
from jax.experimental.pallas import tpu as pltpu
from jax.experimental.pallas import tpu_sc as plsc

_C = 88
_R = 4
_NS = 16

def _make_spmm_pair(nacc, chunks):
    zrows = nacc // _NS
    mesh = plsc.VectorSubcoreMesh(core_axis_name="c", subcore_axis_name="s")

    @functools.partial(
        pl.kernel,
        mesh=mesh,
        out_type=jax.ShapeDtypeStruct((2, nacc, 128), jnp.float32),
        scratch_types=[
            pltpu.VMEM((_R, _C), jnp.int32),
            pltpu.VMEM((_R, _C), jnp.int32),
            pltpu.VMEM((_R, _C, 128), jnp.float32),
            pltpu.VMEM_SHARED((nacc, 128), jnp.float32),
        ] + [pltpu.SemaphoreType.DMA] * (3 * _R),
    )
    def spmm_pair(h_hbm, src_hbm, dst_hbm, zeros_hbm, out_hbm,
                  srcv, dstv, rows, accum, *sems):
        cid = lax.axis_index("c")
        sid = lax.axis_index("s")
        semis = sems[0:_R]
        semid = sems[_R:2 * _R]
        semg = sems[2 * _R:3 * _R]
        pltpu.sync_copy(zeros_hbm, accum.at[pl.ds(sid * zrows, zrows)])
        plsc.subcore_barrier()

        def idx_start(i, r):
            pltpu.async_copy(src_hbm.at[cid, sid, i], srcv.at[r], semis[r])
            pltpu.async_copy(dst_hbm.at[cid, sid, i], dstv.at[r], semid[r])

        def idx_wait(r):
            pltpu.make_async_copy(
                src_hbm.at[cid, sid, 0], srcv.at[r], semis[r]).wait()

        def gather_start(r):
            pltpu.async_copy(h_hbm.at[srcv.at[r]], rows.at[r], semg[r])

        def gather_wait(r):
            pltpu.make_async_copy(
                h_hbm.at[srcv.at[r]], rows.at[r], semg[r]).wait()

        def scatter(r):
            pltpu.make_async_copy(
                src_hbm.at[cid, sid, 0], dstv.at[r], semid[r]).wait()
            pltpu.sync_copy(rows.at[r], accum.at[dstv.at[r]], add=True)

        for r0 in range(_R):
            idx_start(r0, r0)
        for r0 in range(_R - 1):
            idx_wait(r0)
            gather_start(r0)

        def step(g, carry):
            i0 = _R * g
            for r in range(_R):
                i = i0 + r
                nx = (r + _R - 1) % _R
                gather_wait(r)

                @pl.when(i + _R - 1 < chunks)
                def _(i=i, nx=nx):
                    idx_wait(nx)
                    gather_start(nx)

                scatter(r)

                @pl.when(i + _R < chunks)
                def _(i=i, r=r):
                    idx_start(i + _R, r)

            return carry

        lax.fori_loop(0, chunks // _R, step, 0)
        plsc.subcore_barrier()
        pltpu.sync_copy(accum.at[pl.ds(sid * zrows, zrows)],
                        out_hbm.at[cid, pl.ds(sid * zrows, zrows)])

    return spmm_pair


def _dot(a, b):
    return jnp.dot(a, b, preferred_element_type=jnp.float32)


def _stats_body(n, a_ref, cs_ref, csq_ref):
    for half in (0, 1):
        a = a_ref[half, :, :]
        bad = a[n:n + 1, :]
        cs = jnp.sum(a, axis=0, keepdims=True) - bad
        csq = jnp.sum(a * a, axis=0, keepdims=True) - bad * bad
        cs_ref[half, :, :] = jnp.broadcast_to(cs, (8, 128))
        csq_ref[half, :, :] = jnp.broadcast_to(csq, (8, 128))


def _pairnorm_blocks(n, a, cs, csq):
    mu = cs * (1.0 / n)
    t = csq * (1.0 / n) - mu * mu
    rn_a = jnp.sqrt(1e-6 + jnp.sum(t[:, :64]))
    rn_b = jnp.sqrt(1e-6 + jnp.sum(t[:, 64:]))
    c = a - mu
    return jnp.tanh(c[:, :64] / rn_a), jnp.tanh(c[:, 64:] / rn_b)


def _tc0_body(x_ref, w0_ref, w1_ref, out_ref):
    x = x_ref[...]
    out_ref[0, :, :] = _dot(x, w0_ref[...])
    out_ref[1, :, :] = _dot(x, w1_ref[...])


def _tc1_body(n, a_ref, cs_ref, csq_ref, x_ref,
              wx0_ref, wa0_ref, wb0_ref, wx1_ref, wa1_ref, wb1_ref,
              h_ref, b0_ref):
    x = x_ref[...]
    side = ((wx0_ref, wa0_ref, wb0_ref), (wx1_ref, wa1_ref, wb1_ref))
    for half in (0, 1):
        wx, wa, wb = side[half]
        blk_a, blk_b = _pairnorm_blocks(
            n, a_ref[half, :, :], cs_ref[half, 0:1, :], csq_ref[half, 0:1, :])
        h = _dot(x, wx[...]) + _dot(blk_a, wa[...]) + _dot(blk_b, wb[...])
        h_ref[half, :, :] = h
        b0_ref[half, :, :] = jnp.concatenate([blk_a, blk_b], axis=1)


def _tc2_body(n, a_ref, cs_ref, csq_ref, x_ref, b0_ref,
              wx0_ref, wa00_ref, wb00_ref, wa10_ref, wb10_ref,
              wx1_ref, wa01_ref, wb01_ref, wa11_ref, wb11_ref, h_ref):
    x = x_ref[...]
    side = ((wx0_ref, wa00_ref, wb00_ref, wa10_ref, wb10_ref),
            (wx1_ref, wa01_ref, wb01_ref, wa11_ref, wb11_ref))
    for half in (0, 1):
        wx, wa0, wb0, wa1, wb1 = side[half]
        blk_a, blk_b = _pairnorm_blocks(
            n, a_ref[half, :, :], cs_ref[half, 0:1, :], csq_ref[half, 0:1, :])
        b0_a = b0_ref[half, :, :64]
        b0_b = b0_ref[half, :, 64:]
        h = (_dot(x, wx[...]) + _dot(b0_a, wa0[...]) + _dot(b0_b, wb0[...])
             + _dot(blk_a, wa1[...]) + _dot(blk_b, wb1[...]))
        h_ref[half, :, :] = h


def _tc3_body(a_ref, bo1_ref, boc_ref, bo2_ref,
              aw1_ref, ab1_ref, aw2_ref, mw_ref, mb_ref,
              out_ref, beta_ref, emb1_ref, com1_ref, com2_ref, emb2_ref):
    def norm_rows(o):
        nrm = jnp.sqrt(jnp.sum(o * o, axis=1, keepdims=True))
        return o / jnp.maximum(nrm, 1e-12)

    emb1 = norm_rows(a_ref[0, :, :64] + bo1_ref[...])
    com1 = norm_rows(a_ref[0, :, 64:] + boc_ref[...])
    emb2 = norm_rows(a_ref[1, :, :64] + bo2_ref[...])
    com2 = norm_rows(a_ref[1, :, 64:] + boc_ref[...])
    xcom = (com1 + com2) * 0.5

    aw1 = aw1_ref[...]
    ab1 = ab1_ref[...]
    aw2 = aw2_ref[...]
    scores = jnp.concatenate(
        [_dot(jnp.tanh(_dot(v, aw1) + ab1), aw2) for v in (emb1, emb2, xcom)],
        axis=1)
    m = jnp.max(scores, axis=1, keepdims=True)
    ex = jnp.exp(scores - m)
    beta = ex / jnp.sum(ex, axis=1, keepdims=True)

    emb = beta[:, 0:1] * emb1 + beta[:, 1:2] * emb2 + beta[:, 2:3] * xcom
    logits = _dot(emb, mw_ref[...]) + mb_ref[...]
    lm = jnp.max(logits, axis=1, keepdims=True)
    le = jnp.exp(logits - lm)
    out_ref[...] = le / jnp.sum(le, axis=1, keepdims=True)
    beta_ref[...] = beta
    emb1_ref[...] = emb1
    com1_ref[...] = com1
    com2_ref[...] = com2
    emb2_ref[...] = emb2


def _full_spec(shape):
    nd = len(shape)
    return pl.BlockSpec(shape, lambda i, _nd=nd: (0,) * _nd)


def _rows_spec(bs, width):
    return pl.BlockSpec((bs, width), lambda i: (i, 0))


def _half_rows_spec(bs, width):
    return pl.BlockSpec((2, bs, width), lambda i: (0, i, 0))


def kernel(x, params, padj, fadj):
    n, nfeat = x.shape
    e = padj.shape[1]
    f32 = jnp.float32

    chunks = -(-e // (_NS * _C))
    chunks = _R * (-(-chunks // _R))
    t = chunks * _C
    tot = _NS * t
    nacc = _NS * 8 * (-(-(n + 1) // (_NS * 8)))
    bs = nacc // 8
    grid = (8,)

    def prep(src, dst, off):
        s = jnp.pad(src + off, (0, tot - e)).reshape(_NS, chunks, _C)
        d = jnp.pad(dst, (0, tot - e), constant_values=n).reshape(
            _NS, chunks, _C)
        return s, d

    sp, dp = prep(padj[0], padj[1], 0)
    sf, df = prep(fadj[0], fadj[1], n)
    src_all = jnp.stack([sp, sf])
    dst_all = jnp.stack([dp, df])
    zeros = jnp.zeros((nacc // _NS, 128), f32)

    spmm_pair = _make_spmm_pair(nacc, chunks)

    p1, p2, pc = params["sgcn1"], params["sgcn2"], params["cgcn"]
    nh = p1["ws"][1].shape[0] - nfeat
    z64 = jnp.zeros((nh, 64), f32)

    def comb(wa, wb):
        return jnp.concatenate([wa, wb], axis=1)

    w128 = _full_spec((nfeat, 128))
    w64 = _full_spec((nh, 128))
    stat_spec = _full_spec((2, 8, 128))
    stat_shape = jax.ShapeDtypeStruct((2, 8, 128), f32)

    def stats(a):
        return pl.pallas_call(
            functools.partial(_stats_body, n),
            out_shape=[stat_shape, stat_shape],
        )(a)

    h0 = pl.pallas_call(
        _tc0_body,
        grid=grid,
        in_specs=[_rows_spec(bs, nfeat), w128, w128],
        out_specs=_half_rows_spec(bs, 128),
        out_shape=jax.ShapeDtypeStruct((2, n, 128), f32),
    )(x, comb(p1["ws"][0], pc["ws"][0]), comb(p2["ws"][0], pc["ws"][0]))

    a0 = spmm_pair(h0.reshape(2 * n, 128), src_all, dst_all, zeros)

    cs0, csq0 = stats(a0)
    h1, b0 = pl.pallas_call(
        functools.partial(_tc1_body, n),
        grid=grid,
        in_specs=[_half_rows_spec(bs, 128), stat_spec, stat_spec,
                  _rows_spec(bs, nfeat), w128, w64, w64, w128, w64, w64],
        out_specs=[_half_rows_spec(bs, 128), _half_rows_spec(bs, 128)],
        out_shape=[jax.ShapeDtypeStruct((2, n, 128), f32),
                   jax.ShapeDtypeStruct((2, n, 128), f32)],
    )(a0, cs0, csq0, x,
      comb(p1["ws"][1][:nfeat], pc["ws"][1][:nfeat]),
      comb(p1["ws"][1][nfeat:], z64), comb(z64, pc["ws"][1][nfeat:]),
      comb(p2["ws"][1][:nfeat], pc["ws"][1][:nfeat]),
      comb(p2["ws"][1][nfeat:], z64), comb(z64, pc["ws"][1][nfeat:]))

    a1 = spmm_pair(h1.reshape(2 * n, 128), src_all, dst_all, zeros)

    cs1, csq1 = stats(a1)
    h2 = pl.pallas_call(
        functools.partial(_tc2_body, n),
        grid=grid,
        in_specs=[_half_rows_spec(bs, 128), stat_spec, stat_spec,
                  _rows_spec(bs, nfeat), _half_rows_spec(bs, 128),
                  w128, w64, w64, w64, w64, w128, w64, w64, w64, w64],
        out_specs=_half_rows_spec(bs, 128),
        out_shape=jax.ShapeDtypeStruct((2, n, 128), f32),
    )(a1, cs1, csq1, x, b0,
      comb(p1["w_out"][:nfeat], pc["w_out"][:nfeat]),
      comb(p1["w_out"][nfeat:nfeat + nh], z64),
      comb(z64, pc["w_out"][nfeat:nfeat + nh]),
      comb(p1["w_out"][nfeat + nh:], z64),
      comb(z64, pc["w_out"][nfeat + nh:]),
      comb(p2["w_out"][:nfeat], pc["w_out"][:nfeat]),
      comb(p2["w_out"][nfeat:nfeat + nh], z64),
      comb(z64, pc["w_out"][nfeat:nfeat + nh]),
      comb(p2["w_out"][nfeat + nh:], z64),
      comb(z64, pc["w_out"][nfeat + nh:]))

    a2 = spmm_pair(h2.reshape(2 * n, 128), src_all, dst_all, zeros)

    nclass = params["mlp_w"].shape[1]
    out, beta, emb1, com1, com2, emb2 = pl.pallas_call(
        _tc3_body,
        grid=grid,
        in_specs=[_half_rows_spec(bs, 128),
                  _full_spec((64,)), _full_spec((64,)), _full_spec((64,)),
                  _full_spec((64, 2)), _full_spec((2,)), _full_spec((2, 1)),
                  _full_spec((64, nclass)), _full_spec((nclass,))],
        out_specs=[_rows_spec(bs, nclass), _rows_spec(bs, 3),
                   _rows_spec(bs, 64), _rows_spec(bs, 64),
                   _rows_spec(bs, 64), _rows_spec(bs, 64)],
        out_shape=[jax.ShapeDtypeStruct((n, nclass), f32),
                   jax.ShapeDtypeStruct((n, 3), f32),
                   jax.ShapeDtypeStruct((n, 64), f32),
                   jax.ShapeDtypeStruct((n, 64), f32),
                   jax.ShapeDtypeStruct((n, 64), f32),
                   jax.ShapeDtypeStruct((n, 64), f32)],
    )(a2, p1["b_out"], pc["b_out"], p2["b_out"],
      params["att_w1"], params["att_b1"], params["att_w2"],
      params["mlp_w"], params["mlp_b"])

    shift_loss = jnp.zeros((1,), f32)
    return (out, shift_loss, beta.reshape(n, 3, 1), emb1, com1, com2, emb2)


# speedup vs baseline: 1.3650x; 1.0073x over previous
import functools

import jax
import jax.numpy as jnp
import numpy as np
from jax import lax
from jax.experimental import pallas as pl
from jax.experimental.pallas import tpu as pltpu
from jax.experimental.pallas import tpu_sc as plsc

_C = 88
_R = 4
_NS = 16

def _make_spmm_pair(nacc, chunks):
    zrows = nacc // _NS
    mesh = plsc.VectorSubcoreMesh(core_axis_name="c", subcore_axis_name="s")

    @functools.partial(
        pl.kernel,
        mesh=mesh,
        out_type=jax.ShapeDtypeStruct((2, nacc, 128), jnp.float32),
        scratch_types=[
            pltpu.VMEM((_R, _C), jnp.int32),
            pltpu.VMEM((_R, _C), jnp.int32),
            pltpu.VMEM((_R, _C, 128), jnp.float32),
            pltpu.VMEM_SHARED((nacc, 128), jnp.float32),
        ] + [pltpu.SemaphoreType.DMA] * (4 * _R),
    )
    def spmm_pair(h_hbm, src_hbm, dst_hbm, zeros_hbm, out_hbm,
                  srcv, dstv, rows, accum, *sems):
        cid = lax.axis_index("c")
        sid = lax.axis_index("s")
        semis = sems[0:_R]
        semid = sems[_R:2 * _R]
        semg = sems[2 * _R:3 * _R]
        semg2 = sems[3 * _R:4 * _R]
        pltpu.sync_copy(zeros_hbm, accum.at[pl.ds(sid * zrows, zrows)])
        plsc.subcore_barrier()

        def idx_start(i, r):
            pltpu.async_copy(src_hbm.at[cid, sid, i], srcv.at[r], semis[r])
            pltpu.async_copy(dst_hbm.at[cid, sid, i], dstv.at[r], semid[r])

        def idx_wait(r):
            pltpu.make_async_copy(
                src_hbm.at[cid, sid, 0], srcv.at[r], semis[r]).wait()

        h1c = _C - _C // 2 // 8 * 8
        h0c = _C - h1c

        def gather_start(r):
            pltpu.async_copy(h_hbm.at[srcv.at[r, pl.ds(0, h0c)]],
                             rows.at[r, pl.ds(0, h0c)], semg[r])
            pltpu.async_copy(h_hbm.at[srcv.at[r, pl.ds(h0c, h1c)]],
                             rows.at[r, pl.ds(h0c, h1c)], semg2[r])

        def gather_wait(r):
            pltpu.make_async_copy(
                h_hbm.at[srcv.at[r, pl.ds(0, h0c)]],
                rows.at[r, pl.ds(0, h0c)], semg[r]).wait()
            pltpu.make_async_copy(
                h_hbm.at[srcv.at[r, pl.ds(h0c, h1c)]],
                rows.at[r, pl.ds(h0c, h1c)], semg2[r]).wait()

        def scatter(r):
            pltpu.make_async_copy(
                src_hbm.at[cid, sid, 0], dstv.at[r], semid[r]).wait()
            pltpu.sync_copy(rows.at[r], accum.at[dstv.at[r]], add=True)

        for r0 in range(_R):
            idx_start(r0, r0)
        for r0 in range(_R - 1):
            idx_wait(r0)
            gather_start(r0)

        def step(g, carry):
            i0 = _R * g
            for r in range(_R):
                i = i0 + r
                nx = (r + _R - 1) % _R
                gather_wait(r)

                @pl.when(i + _R - 1 < chunks)
                def _(i=i, nx=nx):
                    idx_wait(nx)
                    gather_start(nx)

                scatter(r)

                @pl.when(i + _R < chunks)
                def _(i=i, r=r):
                    idx_start(i + _R, r)

            return carry

        lax.fori_loop(0, chunks // _R, step, 0)
        plsc.subcore_barrier()
        pltpu.sync_copy(accum.at[pl.ds(sid * zrows, zrows)],
                        out_hbm.at[cid, pl.ds(sid * zrows, zrows)])

    return spmm_pair


def _dot(a, b):
    return jnp.dot(a, b, preferred_element_type=jnp.float32)


def _stats_body(n, a_ref, cs_ref, csq_ref):
    for half in (0, 1):
        a = a_ref[half, :, :]
        bad = a[n:n + 1, :]
        cs = jnp.sum(a, axis=0, keepdims=True) - bad
        csq = jnp.sum(a * a, axis=0, keepdims=True) - bad * bad
        cs_ref[half, :, :] = jnp.broadcast_to(cs, (8, 128))
        csq_ref[half, :, :] = jnp.broadcast_to(csq, (8, 128))


def _pairnorm_blocks(n, a, cs, csq):
    mu = cs * (1.0 / n)
    t = csq * (1.0 / n) - mu * mu
    rn_a = jnp.sqrt(1e-6 + jnp.sum(t[:, :64]))
    rn_b = jnp.sqrt(1e-6 + jnp.sum(t[:, 64:]))
    c = a - mu
    return jnp.tanh(c[:, :64] / rn_a), jnp.tanh(c[:, 64:] / rn_b)


def _tc0_body(x_ref, w0_ref, w1_ref, out_ref):
    x = x_ref[...]
    out_ref[0, :, :] = _dot(x, w0_ref[...])
    out_ref[1, :, :] = _dot(x, w1_ref[...])


def _tc1_body(n, a_ref, cs_ref, csq_ref, x_ref,
              wx0_ref, wa0_ref, wb0_ref, wx1_ref, wa1_ref, wb1_ref,
              h_ref, b0_ref):
    x = x_ref[...]
    side = ((wx0_ref, wa0_ref, wb0_ref), (wx1_ref, wa1_ref, wb1_ref))
    for half in (0, 1):
        wx, wa, wb = side[half]
        blk_a, blk_b = _pairnorm_blocks(
            n, a_ref[half, :, :], cs_ref[half, 0:1, :], csq_ref[half, 0:1, :])
        h = _dot(x, wx[...]) + _dot(blk_a, wa[...]) + _dot(blk_b, wb[...])
        h_ref[half, :, :] = h
        b0_ref[half, :, :] = jnp.concatenate([blk_a, blk_b], axis=1)


def _tc2_body(n, a_ref, cs_ref, csq_ref, x_ref, b0_ref,
              wx0_ref, wa00_ref, wb00_ref, wa10_ref, wb10_ref,
              wx1_ref, wa01_ref, wb01_ref, wa11_ref, wb11_ref, h_ref):
    x = x_ref[...]
    side = ((wx0_ref, wa00_ref, wb00_ref, wa10_ref, wb10_ref),
            (wx1_ref, wa01_ref, wb01_ref, wa11_ref, wb11_ref))
    for half in (0, 1):
        wx, wa0, wb0, wa1, wb1 = side[half]
        blk_a, blk_b = _pairnorm_blocks(
            n, a_ref[half, :, :], cs_ref[half, 0:1, :], csq_ref[half, 0:1, :])
        b0_a = b0_ref[half, :, :64]
        b0_b = b0_ref[half, :, 64:]
        h = (_dot(x, wx[...]) + _dot(b0_a, wa0[...]) + _dot(b0_b, wb0[...])
             + _dot(blk_a, wa1[...]) + _dot(blk_b, wb1[...]))
        h_ref[half, :, :] = h


def _tc3_body(a_ref, bo1_ref, boc_ref, bo2_ref,
              aw1_ref, ab1_ref, aw2_ref, mw_ref, mb_ref,
              out_ref, beta_ref, emb1_ref, com1_ref, com2_ref, emb2_ref):
    def norm_rows(o):
        nrm = jnp.sqrt(jnp.sum(o * o, axis=1, keepdims=True))
        return o / jnp.maximum(nrm, 1e-12)

    emb1 = norm_rows(a_ref[0, :, :64] + bo1_ref[...])
    com1 = norm_rows(a_ref[0, :, 64:] + boc_ref[...])
    emb2 = norm_rows(a_ref[1, :, :64] + bo2_ref[...])
    com2 = norm_rows(a_ref[1, :, 64:] + boc_ref[...])
    xcom = (com1 + com2) * 0.5

    aw1 = aw1_ref[...]
    ab1 = ab1_ref[...]
    aw2 = aw2_ref[...]
    scores = jnp.concatenate(
        [_dot(jnp.tanh(_dot(v, aw1) + ab1), aw2) for v in (emb1, emb2, xcom)],
        axis=1)
    m = jnp.max(scores, axis=1, keepdims=True)
    ex = jnp.exp(scores - m)
    beta = ex / jnp.sum(ex, axis=1, keepdims=True)

    emb = beta[:, 0:1] * emb1 + beta[:, 1:2] * emb2 + beta[:, 2:3] * xcom
    logits = _dot(emb, mw_ref[...]) + mb_ref[...]
    lm = jnp.max(logits, axis=1, keepdims=True)
    le = jnp.exp(logits - lm)
    out_ref[...] = le / jnp.sum(le, axis=1, keepdims=True)
    beta_ref[...] = beta
    emb1_ref[...] = emb1
    com1_ref[...] = com1
    com2_ref[...] = com2
    emb2_ref[...] = emb2


def _full_spec(shape):
    nd = len(shape)
    return pl.BlockSpec(shape, lambda i, _nd=nd: (0,) * _nd)


def _rows_spec(bs, width):
    return pl.BlockSpec((bs, width), lambda i: (i, 0))


def _half_rows_spec(bs, width):
    return pl.BlockSpec((2, bs, width), lambda i: (0, i, 0))


def kernel(x, params, padj, fadj):
    n, nfeat = x.shape
    e = padj.shape[1]
    f32 = jnp.float32

    chunks = -(-e // (_NS * _C))
    chunks = _R * (-(-chunks // _R))
    t = chunks * _C
    tot = _NS * t
    nacc = _NS * 8 * (-(-(n + 1) // (_NS * 8)))
    bs = nacc // 8
    grid = (8,)

    def prep(src, dst, off):
        s = jnp.pad(src + off, (0, tot - e)).reshape(_NS, chunks, _C)
        d = jnp.pad(dst, (0, tot - e), constant_values=n).reshape(
            _NS, chunks, _C)
        return s, d

    sp, dp = prep(padj[0], padj[1], 0)
    sf, df = prep(fadj[0], fadj[1], n)
    src_all = jnp.stack([sp, sf])
    dst_all = jnp.stack([dp, df])
    zeros = jnp.zeros((nacc // _NS, 128), f32)

    spmm_pair = _make_spmm_pair(nacc, chunks)

    p1, p2, pc = params["sgcn1"], params["sgcn2"], params["cgcn"]
    nh = p1["ws"][1].shape[0] - nfeat
    z64 = jnp.zeros((nh, 64), f32)

    def comb(wa, wb):
        return jnp.concatenate([wa, wb], axis=1)

    w128 = _full_spec((nfeat, 128))
    w64 = _full_spec((nh, 128))
    stat_spec = _full_spec((2, 8, 128))
    stat_shape = jax.ShapeDtypeStruct((2, 8, 128), f32)

    def stats(a):
        return pl.pallas_call(
            functools.partial(_stats_body, n),
            out_shape=[stat_shape, stat_shape],
        )(a)

    h0 = pl.pallas_call(
        _tc0_body,
        grid=grid,
        in_specs=[_rows_spec(bs, nfeat), w128, w128],
        out_specs=_half_rows_spec(bs, 128),
        out_shape=jax.ShapeDtypeStruct((2, n, 128), f32),
    )(x, comb(p1["ws"][0], pc["ws"][0]), comb(p2["ws"][0], pc["ws"][0]))

    a0 = spmm_pair(h0.reshape(2 * n, 128), src_all, dst_all, zeros)

    cs0, csq0 = stats(a0)
    h1, b0 = pl.pallas_call(
        functools.partial(_tc1_body, n),
        grid=grid,
        in_specs=[_half_rows_spec(bs, 128), stat_spec, stat_spec,
                  _rows_spec(bs, nfeat), w128, w64, w64, w128, w64, w64],
        out_specs=[_half_rows_spec(bs, 128), _half_rows_spec(bs, 128)],
        out_shape=[jax.ShapeDtypeStruct((2, n, 128), f32),
                   jax.ShapeDtypeStruct((2, n, 128), f32)],
    )(a0, cs0, csq0, x,
      comb(p1["ws"][1][:nfeat], pc["ws"][1][:nfeat]),
      comb(p1["ws"][1][nfeat:], z64), comb(z64, pc["ws"][1][nfeat:]),
      comb(p2["ws"][1][:nfeat], pc["ws"][1][:nfeat]),
      comb(p2["ws"][1][nfeat:], z64), comb(z64, pc["ws"][1][nfeat:]))

    a1 = spmm_pair(h1.reshape(2 * n, 128), src_all, dst_all, zeros)

    cs1, csq1 = stats(a1)
    h2 = pl.pallas_call(
        functools.partial(_tc2_body, n),
        grid=grid,
        in_specs=[_half_rows_spec(bs, 128), stat_spec, stat_spec,
                  _rows_spec(bs, nfeat), _half_rows_spec(bs, 128),
                  w128, w64, w64, w64, w64, w128, w64, w64, w64, w64],
        out_specs=_half_rows_spec(bs, 128),
        out_shape=jax.ShapeDtypeStruct((2, n, 128), f32),
    )(a1, cs1, csq1, x, b0,
      comb(p1["w_out"][:nfeat], pc["w_out"][:nfeat]),
      comb(p1["w_out"][nfeat:nfeat + nh], z64),
      comb(z64, pc["w_out"][nfeat:nfeat + nh]),
      comb(p1["w_out"][nfeat + nh:], z64),
      comb(z64, pc["w_out"][nfeat + nh:]),
      comb(p2["w_out"][:nfeat], pc["w_out"][:nfeat]),
      comb(p2["w_out"][nfeat:nfeat + nh], z64),
      comb(z64, pc["w_out"][nfeat:nfeat + nh]),
      comb(p2["w_out"][nfeat + nh:], z64),
      comb(z64, pc["w_out"][nfeat + nh:]))

    a2 = spmm_pair(h2.reshape(2 * n, 128), src_all, dst_all, zeros)

    nclass = params["mlp_w"].shape[1]
    out, beta, emb1, com1, com2, emb2 = pl.pallas_call(
        _tc3_body,
        grid=grid,
        in_specs=[_half_rows_spec(bs, 128),
                  _full_spec((64,)), _full_spec((64,)), _full_spec((64,)),
                  _full_spec((64, 2)), _full_spec((2,)), _full_spec((2, 1)),
                  _full_spec((64, nclass)), _full_spec((nclass,))],
        out_specs=[_rows_spec(bs, nclass), _rows_spec(bs, 3),
                   _rows_spec(bs, 64), _rows_spec(bs, 64),
                   _rows_spec(bs, 64), _rows_spec(bs, 64)],
        out_shape=[jax.ShapeDtypeStruct((n, nclass), f32),
                   jax.ShapeDtypeStruct((n, 3), f32),
                   jax.ShapeDtypeStruct((n, 64), f32),
                   jax.ShapeDtypeStruct((n, 64), f32),
                   jax.ShapeDtypeStruct((n, 64), f32),
                   jax.ShapeDtypeStruct((n, 64), f32)],
    )(a2, p1["b_out"], pc["b_out"], p2["b_out"],
      params["att_w1"], params["att_b1"], params["att_w2"],
      params["mlp_w"], params["mlp_b"])

    shift_loss = jnp.zeros((1,), f32)
    return (out, shift_loss, beta.reshape(n, 3, 1), emb1, com1, com2, emb2)


# speedup vs baseline: 1.3674x; 1.0018x over previous
import functools

import jax
import jax.numpy as jnp
from jax import lax
from jax.experimental import pallas as pl
from jax.experimental.pallas import tpu as pltpu
from jax.experimental.pallas import tpu_sc as plsc

_C = 88
_R = 4
_NS = 16

def _make_spmm_pair(nacc, chunks):
    zrows = nacc // _NS
    mesh = plsc.VectorSubcoreMesh(core_axis_name="c", subcore_axis_name="s")

    @functools.partial(
        pl.kernel,
        mesh=mesh,
        out_type=jax.ShapeDtypeStruct((2, nacc, 128), jnp.float32),
        scratch_types=[
            pltpu.VMEM((_R, _C), jnp.int32),
            pltpu.VMEM((_R, _C), jnp.int32),
            pltpu.VMEM((_R, _C, 128), jnp.float32),
            pltpu.VMEM_SHARED((nacc, 128), jnp.float32),
        ] + [pltpu.SemaphoreType.DMA] * (4 * _R),
    )
    def spmm_pair(h_hbm, src_hbm, dst_hbm, zeros_hbm, out_hbm,
                  srcv, dstv, rows, accum, *sems):
        cid = lax.axis_index("c")
        sid = lax.axis_index("s")
        semis = sems[0:_R]
        semid = sems[_R:2 * _R]
        semg = sems[2 * _R:3 * _R]
        semg2 = sems[3 * _R:4 * _R]
        pltpu.sync_copy(zeros_hbm, accum.at[pl.ds(sid * zrows, zrows)])
        plsc.subcore_barrier()

        def idx_start(i, r):
            pltpu.async_copy(src_hbm.at[cid, sid, i], srcv.at[r], semis[r])
            pltpu.async_copy(dst_hbm.at[cid, sid, i], dstv.at[r], semid[r])

        def idx_wait(r):
            pltpu.make_async_copy(
                src_hbm.at[cid, sid, 0], srcv.at[r], semis[r]).wait()

        h1c = _C - _C // 2 // 8 * 8
        h0c = _C - h1c

        def gather_start(r):
            pltpu.async_copy(h_hbm.at[srcv.at[r, pl.ds(0, h0c)]],
                             rows.at[r, pl.ds(0, h0c)], semg[r])
            pltpu.async_copy(h_hbm.at[srcv.at[r, pl.ds(h0c, h1c)]],
                             rows.at[r, pl.ds(h0c, h1c)], semg2[r])

        def gather_wait(r):
            pltpu.make_async_copy(
                h_hbm.at[srcv.at[r, pl.ds(0, h0c)]],
                rows.at[r, pl.ds(0, h0c)], semg[r]).wait()
            pltpu.make_async_copy(
                h_hbm.at[srcv.at[r, pl.ds(h0c, h1c)]],
                rows.at[r, pl.ds(h0c, h1c)], semg2[r]).wait()

        def scatter(r):
            pltpu.make_async_copy(
                src_hbm.at[cid, sid, 0], dstv.at[r], semid[r]).wait()
            pltpu.sync_copy(rows.at[r], accum.at[dstv.at[r]], add=True)

        for r0 in range(_R):
            idx_start(r0, r0)
        for r0 in range(_R - 1):
            idx_wait(r0)
            gather_start(r0)

        def step(g, carry):
            i0 = _R * g
            for r in range(_R):
                i = i0 + r
                nx = (r + _R - 1) % _R
                gather_wait(r)

                @pl.when(i + _R - 1 < chunks)
                def _(i=i, nx=nx):
                    idx_wait(nx)
                    gather_start(nx)

                scatter(r)

                @pl.when(i + _R < chunks)
                def _(i=i, r=r):
                    idx_start(i + _R, r)

            return carry

        lax.fori_loop(0, chunks // _R, step, 0)
        plsc.subcore_barrier()
        pltpu.sync_copy(accum.at[pl.ds(sid * zrows, zrows)],
                        out_hbm.at[cid, pl.ds(sid * zrows, zrows)])

    return spmm_pair


def _dot(a, b):
    return jnp.dot(a, b, preferred_element_type=jnp.float32)


def _stats_body(n, a_ref, cs_ref, csq_ref):
    for half in (0, 1):
        a = a_ref[half, :, :]
        bad = a[n:n + 1, :]
        cs = jnp.sum(a, axis=0, keepdims=True) - bad
        csq = jnp.sum(a * a, axis=0, keepdims=True) - bad * bad
        cs_ref[half, :, :] = jnp.broadcast_to(cs, (8, 128))
        csq_ref[half, :, :] = jnp.broadcast_to(csq, (8, 128))


def _pairnorm_blocks(n, a, cs, csq):
    mu = cs * (1.0 / n)
    t = csq * (1.0 / n) - mu * mu
    rn_a = jnp.sqrt(1e-6 + jnp.sum(t[:, :64]))
    rn_b = jnp.sqrt(1e-6 + jnp.sum(t[:, 64:]))
    c = a - mu
    return jnp.tanh(c[:, :64] / rn_a), jnp.tanh(c[:, 64:] / rn_b)


def _tc0_body(x_ref, w0_ref, w1_ref, out_ref):
    x = x_ref[...]
    out_ref[0, :, :] = _dot(x, w0_ref[...])
    out_ref[1, :, :] = _dot(x, w1_ref[...])


def _tc1_body(n, a_ref, cs_ref, csq_ref, x_ref,
              wx0_ref, wa0_ref, wb0_ref, wx1_ref, wa1_ref, wb1_ref,
              h_ref, b0_ref):
    x = x_ref[...]
    side = ((wx0_ref, wa0_ref, wb0_ref), (wx1_ref, wa1_ref, wb1_ref))
    for half in (0, 1):
        wx, wa, wb = side[half]
        blk_a, blk_b = _pairnorm_blocks(
            n, a_ref[half, :, :], cs_ref[half, 0:1, :], csq_ref[half, 0:1, :])
        h = _dot(x, wx[...]) + _dot(blk_a, wa[...]) + _dot(blk_b, wb[...])
        h_ref[half, :, :] = h
        b0_ref[half, :, :] = jnp.concatenate([blk_a, blk_b], axis=1)


def _tc2_body(n, a_ref, cs_ref, csq_ref, x_ref, b0_ref,
              wx0_ref, wa00_ref, wb00_ref, wa10_ref, wb10_ref,
              wx1_ref, wa01_ref, wb01_ref, wa11_ref, wb11_ref, h_ref):
    x = x_ref[...]
    side = ((wx0_ref, wa00_ref, wb00_ref, wa10_ref, wb10_ref),
            (wx1_ref, wa01_ref, wb01_ref, wa11_ref, wb11_ref))
    for half in (0, 1):
        wx, wa0, wb0, wa1, wb1 = side[half]
        blk_a, blk_b = _pairnorm_blocks(
            n, a_ref[half, :, :], cs_ref[half, 0:1, :], csq_ref[half, 0:1, :])
        b0_a = b0_ref[half, :, :64]
        b0_b = b0_ref[half, :, 64:]
        h = (_dot(x, wx[...]) + _dot(b0_a, wa0[...]) + _dot(b0_b, wb0[...])
             + _dot(blk_a, wa1[...]) + _dot(blk_b, wb1[...]))
        h_ref[half, :, :] = h


def _tc3_body(a_ref, bo1_ref, boc_ref, bo2_ref,
              aw1_ref, ab1_ref, aw2_ref, mw_ref, mb_ref,
              out_ref, beta_ref, emb1_ref, com1_ref, com2_ref, emb2_ref):
    def norm_rows(o):
        nrm = jnp.sqrt(jnp.sum(o * o, axis=1, keepdims=True))
        return o / jnp.maximum(nrm, 1e-12)

    emb1 = norm_rows(a_ref[0, :, :64] + bo1_ref[...])
    com1 = norm_rows(a_ref[0, :, 64:] + boc_ref[...])
    emb2 = norm_rows(a_ref[1, :, :64] + bo2_ref[...])
    com2 = norm_rows(a_ref[1, :, 64:] + boc_ref[...])
    xcom = (com1 + com2) * 0.5

    aw1 = aw1_ref[...]
    ab1 = ab1_ref[...]
    aw2 = aw2_ref[...]
    scores = jnp.concatenate(
        [_dot(jnp.tanh(_dot(v, aw1) + ab1), aw2) for v in (emb1, emb2, xcom)],
        axis=1)
    m = jnp.max(scores, axis=1, keepdims=True)
    ex = jnp.exp(scores - m)
    beta = ex / jnp.sum(ex, axis=1, keepdims=True)

    emb = beta[:, 0:1] * emb1 + beta[:, 1:2] * emb2 + beta[:, 2:3] * xcom
    logits = _dot(emb, mw_ref[...]) + mb_ref[...]
    lm = jnp.max(logits, axis=1, keepdims=True)
    le = jnp.exp(logits - lm)
    out_ref[...] = le / jnp.sum(le, axis=1, keepdims=True)
    beta_ref[...] = beta
    emb1_ref[...] = emb1
    com1_ref[...] = com1
    com2_ref[...] = com2
    emb2_ref[...] = emb2


def _full_spec(shape):
    nd = len(shape)
    return pl.BlockSpec(shape, lambda i, _nd=nd: (0,) * _nd)


def _rows_spec(bs, width):
    return pl.BlockSpec((bs, width), lambda i: (i, 0))


def _half_rows_spec(bs, width):
    return pl.BlockSpec((2, bs, width), lambda i: (0, i, 0))


def kernel(x, params, padj, fadj):
    n, nfeat = x.shape
    e = padj.shape[1]
    f32 = jnp.float32

    chunks = -(-e // (_NS * _C))
    chunks = _R * (-(-chunks // _R))
    t = chunks * _C
    tot = _NS * t
    nacc = _NS * 8 * (-(-(n + 1) // (_NS * 8)))
    bs = nacc // 8
    grid = (8,)

    def prep(src, dst, off):
        s = jnp.pad(src + off, (0, tot - e)).reshape(_NS, chunks, _C)
        d = jnp.pad(dst, (0, tot - e), constant_values=n).reshape(
            _NS, chunks, _C)
        return s, d

    sp, dp = prep(padj[0], padj[1], 0)
    sf, df = prep(fadj[0], fadj[1], n)
    src_all = jnp.stack([sp, sf])
    dst_all = jnp.stack([dp, df])
    zeros = jnp.zeros((nacc // _NS, 128), f32)

    spmm_pair = _make_spmm_pair(nacc, chunks)

    p1, p2, pc = params["sgcn1"], params["sgcn2"], params["cgcn"]
    nh = p1["ws"][1].shape[0] - nfeat
    z64 = jnp.zeros((nh, 64), f32)

    def comb(wa, wb):
        return jnp.concatenate([wa, wb], axis=1)

    w128 = _full_spec((nfeat, 128))
    w64 = _full_spec((nh, 128))
    stat_spec = _full_spec((2, 8, 128))
    stat_shape = jax.ShapeDtypeStruct((2, 8, 128), f32)

    def stats(a):
        return pl.pallas_call(
            functools.partial(_stats_body, n),
            out_shape=[stat_shape, stat_shape],
        )(a)

    h0 = pl.pallas_call(
        _tc0_body,
        grid=grid,
        in_specs=[_rows_spec(bs, nfeat), w128, w128],
        out_specs=_half_rows_spec(bs, 128),
        out_shape=jax.ShapeDtypeStruct((2, n, 128), f32),
    )(x, comb(p1["ws"][0], pc["ws"][0]), comb(p2["ws"][0], pc["ws"][0]))

    a0 = spmm_pair(h0.reshape(2 * n, 128), src_all, dst_all, zeros)

    cs0, csq0 = stats(a0)
    h1, b0 = pl.pallas_call(
        functools.partial(_tc1_body, n),
        grid=grid,
        in_specs=[_half_rows_spec(bs, 128), stat_spec, stat_spec,
                  _rows_spec(bs, nfeat), w128, w64, w64, w128, w64, w64],
        out_specs=[_half_rows_spec(bs, 128), _half_rows_spec(bs, 128)],
        out_shape=[jax.ShapeDtypeStruct((2, n, 128), f32),
                   jax.ShapeDtypeStruct((2, n, 128), f32)],
    )(a0, cs0, csq0, x,
      comb(p1["ws"][1][:nfeat], pc["ws"][1][:nfeat]),
      comb(p1["ws"][1][nfeat:], z64), comb(z64, pc["ws"][1][nfeat:]),
      comb(p2["ws"][1][:nfeat], pc["ws"][1][:nfeat]),
      comb(p2["ws"][1][nfeat:], z64), comb(z64, pc["ws"][1][nfeat:]))

    a1 = spmm_pair(h1.reshape(2 * n, 128), src_all, dst_all, zeros)

    cs1, csq1 = stats(a1)
    h2 = pl.pallas_call(
        functools.partial(_tc2_body, n),
        grid=grid,
        in_specs=[_half_rows_spec(bs, 128), stat_spec, stat_spec,
                  _rows_spec(bs, nfeat), _half_rows_spec(bs, 128),
                  w128, w64, w64, w64, w64, w128, w64, w64, w64, w64],
        out_specs=_half_rows_spec(bs, 128),
        out_shape=jax.ShapeDtypeStruct((2, n, 128), f32),
    )(a1, cs1, csq1, x, b0,
      comb(p1["w_out"][:nfeat], pc["w_out"][:nfeat]),
      comb(p1["w_out"][nfeat:nfeat + nh], z64),
      comb(z64, pc["w_out"][nfeat:nfeat + nh]),
      comb(p1["w_out"][nfeat + nh:], z64),
      comb(z64, pc["w_out"][nfeat + nh:]),
      comb(p2["w_out"][:nfeat], pc["w_out"][:nfeat]),
      comb(p2["w_out"][nfeat:nfeat + nh], z64),
      comb(z64, pc["w_out"][nfeat:nfeat + nh]),
      comb(p2["w_out"][nfeat + nh:], z64),
      comb(z64, pc["w_out"][nfeat + nh:]))

    a2 = spmm_pair(h2.reshape(2 * n, 128), src_all, dst_all, zeros)

    nclass = params["mlp_w"].shape[1]
    out, beta, emb1, com1, com2, emb2 = pl.pallas_call(
        _tc3_body,
        grid=grid,
        in_specs=[_half_rows_spec(bs, 128),
                  _full_spec((64,)), _full_spec((64,)), _full_spec((64,)),
                  _full_spec((64, 2)), _full_spec((2,)), _full_spec((2, 1)),
                  _full_spec((64, nclass)), _full_spec((nclass,))],
        out_specs=[_rows_spec(bs, nclass), _rows_spec(bs, 3),
                   _rows_spec(bs, 64), _rows_spec(bs, 64),
                   _rows_spec(bs, 64), _rows_spec(bs, 64)],
        out_shape=[jax.ShapeDtypeStruct((n, nclass), f32),
                   jax.ShapeDtypeStruct((n, 3), f32),
                   jax.ShapeDtypeStruct((n, 64), f32),
                   jax.ShapeDtypeStruct((n, 64), f32),
                   jax.ShapeDtypeStruct((n, 64), f32),
                   jax.ShapeDtypeStruct((n, 64), f32)],
    )(a2, p1["b_out"], pc["b_out"], p2["b_out"],
      params["att_w1"], params["att_b1"], params["att_w2"],
      params["mlp_w"], params["mlp_b"])

    shift_loss = jnp.zeros((1,), f32)
    return (out, shift_loss, beta.reshape(n, 3, 1), emb1, com1, com2, emb2)
